# baseline (device time: 405220 ns/iter reference)
import jax
import jax.numpy as jnp
from jax import lax
from jax.experimental import pallas as pl
from jax.experimental.pallas import tpu as pltpu

N_DEV = 4


def kernel(x):
    xb = x.astype(jnp.bfloat16)
    m, n = xb.shape
    ncol = n // N_DEV

    def body(x_ref, out_ref, local_sem, send_sems, recv_sems):
        my = lax.axis_index("i")

        barrier_sem = pltpu.get_barrier_semaphore()
        for d in range(1, N_DEV):
            peer = lax.rem(my + d, N_DEV)
            pl.semaphore_signal(
                barrier_sem, inc=1,
                device_id=(peer,), device_id_type=pl.DeviceIdType.MESH,
            )
        pl.semaphore_wait(barrier_sem, N_DEV - 1)

        local = pltpu.make_async_copy(
            x_ref.at[:, pl.ds(my * ncol, ncol)],
            out_ref.at[pl.ds(my * m, m), :],
            local_sem,
        )
        local.start()

        def make_rdma(d):
            dst = lax.rem(my + d, N_DEV)
            return pltpu.make_async_remote_copy(
                src_ref=x_ref.at[:, pl.ds(dst * ncol, ncol)],
                dst_ref=out_ref.at[pl.ds(my * m, m), :],
                send_sem=send_sems.at[d - 1],
                recv_sem=recv_sems.at[d - 1],
                device_id=(dst,),
                device_id_type=pl.DeviceIdType.MESH,
            )

        def make_recv(d):
            src_dev = lax.rem(my - d + N_DEV, N_DEV)
            return pltpu.make_async_remote_copy(
                src_ref=x_ref.at[:, pl.ds(src_dev * ncol, ncol)],
                dst_ref=out_ref.at[pl.ds(src_dev * m, m), :],
                send_sem=send_sems.at[d - 1],
                recv_sem=recv_sems.at[d - 1],
                device_id=(src_dev,),
                device_id_type=pl.DeviceIdType.MESH,
            )

        right = make_rdma(1)
        left = make_rdma(3)
        right.start()
        left.start()
        local.wait()
        right.wait_send()
        left.wait_send()
        make_recv(1).wait_recv()
        make_recv(3).wait_recv()

        diag = make_rdma(2)
        diag.start()
        diag.wait_send()
        make_recv(2).wait_recv()

    return pl.pallas_call(
        body,
        out_shape=jax.ShapeDtypeStruct((N_DEV * m, ncol), xb.dtype),
        in_specs=[pl.BlockSpec(memory_space=pl.ANY)],
        out_specs=pl.BlockSpec(memory_space=pl.ANY),
        scratch_shapes=[
            pltpu.SemaphoreType.DMA,
            pltpu.SemaphoreType.DMA((N_DEV - 1,)),
            pltpu.SemaphoreType.DMA((N_DEV - 1,)),
        ],
        compiler_params=pltpu.CompilerParams(collective_id=0),
    )(xb)


# device time: 210626 ns/iter; 1.9239x vs baseline; 1.9239x over previous
import jax
import jax.numpy as jnp
from jax import lax
from jax.experimental import pallas as pl
from jax.experimental.pallas import tpu as pltpu

N_DEV = 4
N_CHUNK = 8


def kernel(x):
    m, n = x.shape
    ncol = n // N_DEV
    c = m // N_CHUNK

    def body(x_ref, out_ref, fbuf, bbuf, copy_sems, local_sems,
             send_sems, recv_sems):
        my = lax.axis_index("i")

        barrier_sem = pltpu.get_barrier_semaphore()
        for d in range(1, N_DEV):
            peer = lax.rem(my + d, N_DEV)
            pl.semaphore_signal(
                barrier_sem, inc=1,
                device_id=(peer,), device_id_type=pl.DeviceIdType.MESH,
            )
        pl.semaphore_wait(barrier_sem, N_DEV - 1)

        def start_chunk_in(k):
            cp = pltpu.make_async_copy(
                x_ref.at[pl.ds(k * c, c), :],
                fbuf.at[k % 2],
                copy_sems.at[k % 2],
            )
            cp.start()
            return cp

        inflight = [start_chunk_in(0), start_chunk_in(1)]

        sends = []
        locals_ = []
        for k in range(N_CHUNK):
            inflight[k % 2].wait()
            bbuf[pl.ds(k * c, c), :] = fbuf[k % 2].astype(jnp.bfloat16)
            if k + 2 < N_CHUNK:
                inflight[k % 2] = start_chunk_in(k + 2)

            lc = pltpu.make_async_copy(
                bbuf.at[pl.ds(k * c, c), pl.ds(my * ncol, ncol)],
                out_ref.at[pl.ds(my * m + k * c, c), :],
                local_sems.at[k],
            )
            lc.start()
            locals_.append(lc)

            for d in range(1, N_DEV):
                dst = lax.rem(my + d, N_DEV)
                rdma = pltpu.make_async_remote_copy(
                    src_ref=bbuf.at[pl.ds(k * c, c), pl.ds(dst * ncol, ncol)],
                    dst_ref=out_ref.at[pl.ds(my * m + k * c, c), :],
                    send_sem=send_sems.at[d - 1, k],
                    recv_sem=recv_sems.at[d - 1, k],
                    device_id=(dst,),
                    device_id_type=pl.DeviceIdType.MESH,
                )
                rdma.start()
                sends.append(rdma)

        for lc in locals_:
            lc.wait()
        for rdma in sends:
            rdma.wait_send()

        for d in range(1, N_DEV):
            src_dev = lax.rem(my - d + N_DEV, N_DEV)
            for k in range(N_CHUNK):
                recv = pltpu.make_async_remote_copy(
                    src_ref=bbuf.at[pl.ds(k * c, c), pl.ds(src_dev * ncol, ncol)],
                    dst_ref=out_ref.at[pl.ds(src_dev * m + k * c, c), :],
                    send_sem=send_sems.at[d - 1, k],
                    recv_sem=recv_sems.at[d - 1, k],
                    device_id=(src_dev,),
                    device_id_type=pl.DeviceIdType.MESH,
                )
                recv.wait_recv()

    return pl.pallas_call(
        body,
        out_shape=jax.ShapeDtypeStruct((N_DEV * m, ncol), jnp.bfloat16),
        in_specs=[pl.BlockSpec(memory_space=pl.ANY)],
        out_specs=pl.BlockSpec(memory_space=pl.ANY),
        scratch_shapes=[
            pltpu.VMEM((2, c, n), x.dtype),
            pltpu.VMEM((m, n), jnp.bfloat16),
            pltpu.SemaphoreType.DMA((2,)),
            pltpu.SemaphoreType.DMA((N_CHUNK,)),
            pltpu.SemaphoreType.DMA((N_DEV - 1, N_CHUNK)),
            pltpu.SemaphoreType.DMA((N_DEV - 1, N_CHUNK)),
        ],
        compiler_params=pltpu.CompilerParams(
            collective_id=0,
            vmem_limit_bytes=100 * 1024 * 1024,
        ),
    )(x)


# device time: 209650 ns/iter; 1.9328x vs baseline; 1.0047x over previous
import jax
import jax.numpy as jnp
from jax import lax
from jax.experimental import pallas as pl
from jax.experimental.pallas import tpu as pltpu

N_DEV = 4
N_CHUNK = 16


def kernel(x):
    m, n = x.shape
    ncol = n // N_DEV
    c = m // N_CHUNK

    def body(x_ref, out_ref, fbuf, bbuf, copy_sems, local_sems,
             send_sems, recv_sems):
        my = lax.axis_index("i")

        barrier_sem = pltpu.get_barrier_semaphore()
        for d in range(1, N_DEV):
            peer = lax.rem(my + d, N_DEV)
            pl.semaphore_signal(
                barrier_sem, inc=1,
                device_id=(peer,), device_id_type=pl.DeviceIdType.MESH,
            )
        pl.semaphore_wait(barrier_sem, N_DEV - 1)

        def start_chunk_in(k):
            cp = pltpu.make_async_copy(
                x_ref.at[pl.ds(k * c, c), :],
                fbuf.at[k % 2],
                copy_sems.at[k % 2],
            )
            cp.start()
            return cp

        inflight = [start_chunk_in(0), start_chunk_in(1)]

        sends = []
        locals_ = []
        for k in range(N_CHUNK):
            inflight[k % 2].wait()
            bbuf[pl.ds(k * c, c), :] = fbuf[k % 2].astype(jnp.bfloat16)
            if k + 2 < N_CHUNK:
                inflight[k % 2] = start_chunk_in(k + 2)

            lc = pltpu.make_async_copy(
                bbuf.at[pl.ds(k * c, c), pl.ds(my * ncol, ncol)],
                out_ref.at[pl.ds(my * m + k * c, c), :],
                local_sems.at[k],
            )
            lc.start()
            locals_.append(lc)

            for d in range(1, N_DEV):
                dst = lax.rem(my + d, N_DEV)
                rdma = pltpu.make_async_remote_copy(
                    src_ref=bbuf.at[pl.ds(k * c, c), pl.ds(dst * ncol, ncol)],
                    dst_ref=out_ref.at[pl.ds(my * m + k * c, c), :],
                    send_sem=send_sems.at[d - 1, k],
                    recv_sem=recv_sems.at[d - 1, k],
                    device_id=(dst,),
                    device_id_type=pl.DeviceIdType.MESH,
                )
                rdma.start()
                sends.append(rdma)

        for lc in locals_:
            lc.wait()
        for rdma in sends:
            rdma.wait_send()

        for d in range(1, N_DEV):
            src_dev = lax.rem(my - d + N_DEV, N_DEV)
            for k in range(N_CHUNK):
                recv = pltpu.make_async_remote_copy(
                    src_ref=bbuf.at[pl.ds(k * c, c), pl.ds(src_dev * ncol, ncol)],
                    dst_ref=out_ref.at[pl.ds(src_dev * m + k * c, c), :],
                    send_sem=send_sems.at[d - 1, k],
                    recv_sem=recv_sems.at[d - 1, k],
                    device_id=(src_dev,),
                    device_id_type=pl.DeviceIdType.MESH,
                )
                recv.wait_recv()

    return pl.pallas_call(
        body,
        out_shape=jax.ShapeDtypeStruct((N_DEV * m, ncol), jnp.bfloat16),
        in_specs=[pl.BlockSpec(memory_space=pl.ANY)],
        out_specs=pl.BlockSpec(memory_space=pl.ANY),
        scratch_shapes=[
            pltpu.VMEM((2, c, n), x.dtype),
            pltpu.VMEM((m, n), jnp.bfloat16),
            pltpu.SemaphoreType.DMA((2,)),
            pltpu.SemaphoreType.DMA((N_CHUNK,)),
            pltpu.SemaphoreType.DMA((N_DEV - 1, N_CHUNK)),
            pltpu.SemaphoreType.DMA((N_DEV - 1, N_CHUNK)),
        ],
        compiler_params=pltpu.CompilerParams(
            collective_id=0,
            vmem_limit_bytes=100 * 1024 * 1024,
        ),
    )(x)


# device time: 208395 ns/iter; 1.9445x vs baseline; 1.0060x over previous
import jax
import jax.numpy as jnp
from jax import lax
from jax.experimental import pallas as pl
from jax.experimental.pallas import tpu as pltpu

N_DEV = 4
N_CHUNK = 16


def kernel(x):
    m, n = x.shape
    ncol = n // N_DEV
    c = m // N_CHUNK

    def body(x_ref, out_ref, fbuf, bbuf, copy_sems, local_sems,
             send_sems, recv_sems):
        my = lax.axis_index("i")

        barrier_sem = pltpu.get_barrier_semaphore()
        for d in range(1, N_DEV):
            peer = lax.rem(my + d, N_DEV)
            pl.semaphore_signal(
                barrier_sem, inc=1,
                device_id=(peer,), device_id_type=pl.DeviceIdType.MESH,
            )

        def start_chunk_in(k):
            cp = pltpu.make_async_copy(
                x_ref.at[pl.ds(k * c, c), :],
                fbuf.at[k % 2],
                copy_sems.at[k % 2],
            )
            cp.start()
            return cp

        inflight = [start_chunk_in(0), start_chunk_in(1)]

        sends = []
        locals_ = []
        for k in range(N_CHUNK):
            inflight[k % 2].wait()
            bbuf[pl.ds(k * c, c), :] = fbuf[k % 2].astype(jnp.bfloat16)
            if k + 2 < N_CHUNK:
                inflight[k % 2] = start_chunk_in(k + 2)

            lc = pltpu.make_async_copy(
                bbuf.at[pl.ds(k * c, c), pl.ds(my * ncol, ncol)],
                out_ref.at[pl.ds(my * m + k * c, c), :],
                local_sems.at[k],
            )
            lc.start()
            locals_.append(lc)

            if k == 0:
                pl.semaphore_wait(barrier_sem, N_DEV - 1)

            for d in range(1, N_DEV):
                dst = lax.rem(my + d, N_DEV)
                rdma = pltpu.make_async_remote_copy(
                    src_ref=bbuf.at[pl.ds(k * c, c), pl.ds(dst * ncol, ncol)],
                    dst_ref=out_ref.at[pl.ds(my * m + k * c, c), :],
                    send_sem=send_sems.at[d - 1, k],
                    recv_sem=recv_sems.at[d - 1, k],
                    device_id=(dst,),
                    device_id_type=pl.DeviceIdType.MESH,
                )
                rdma.start()
                sends.append(rdma)

        for lc in locals_:
            lc.wait()
        for rdma in sends:
            rdma.wait_send()

        for d in range(1, N_DEV):
            src_dev = lax.rem(my - d + N_DEV, N_DEV)
            for k in range(N_CHUNK):
                recv = pltpu.make_async_remote_copy(
                    src_ref=bbuf.at[pl.ds(k * c, c), pl.ds(src_dev * ncol, ncol)],
                    dst_ref=out_ref.at[pl.ds(src_dev * m + k * c, c), :],
                    send_sem=send_sems.at[d - 1, k],
                    recv_sem=recv_sems.at[d - 1, k],
                    device_id=(src_dev,),
                    device_id_type=pl.DeviceIdType.MESH,
                )
                recv.wait_recv()

    return pl.pallas_call(
        body,
        out_shape=jax.ShapeDtypeStruct((N_DEV * m, ncol), jnp.bfloat16),
        in_specs=[pl.BlockSpec(memory_space=pl.ANY)],
        out_specs=pl.BlockSpec(memory_space=pl.ANY),
        scratch_shapes=[
            pltpu.VMEM((2, c, n), x.dtype),
            pltpu.VMEM((m, n), jnp.bfloat16),
            pltpu.SemaphoreType.DMA((2,)),
            pltpu.SemaphoreType.DMA((N_CHUNK,)),
            pltpu.SemaphoreType.DMA((N_DEV - 1, N_CHUNK)),
            pltpu.SemaphoreType.DMA((N_DEV - 1, N_CHUNK)),
        ],
        compiler_params=pltpu.CompilerParams(
            collective_id=0,
            vmem_limit_bytes=100 * 1024 * 1024,
        ),
    )(x)
